# baseline (speedup 1.0000x reference)
import functools

import jax
import jax.numpy as jnp
from jax import lax
from jax.experimental import pallas as pl
from jax.experimental.pallas import tpu as pltpu
from jax.experimental.pallas import tpu_sc as plsc

_TBL = 32
_NTBL = _TBL * _TBL


def _build_rating_table(e_v, r_v, n_stim, n_dim):
    def vec_body(v, _):
        ent = lax.iota(jnp.int32, 16) + v * 16
        i0 = jnp.minimum(ent >> 5, n_stim - 1)
        i1 = jnp.minimum(ent & (_TBL - 1), n_stim - 1)
        acc = jnp.zeros((16,), jnp.float32)
        for d in range(n_dim):
            dcol = jnp.full((16,), d, jnp.int32)
            z0 = plsc.load_gather(e_v, [i0, dcol])
            z1 = plsc.load_gather(e_v, [i1, dcol])
            df = z0 - z1
            acc = acc + df * df
        y = plsc.bitcast((plsc.bitcast(acc, jnp.int32) >> 1) + 0x1FBD1DF5,
                         jnp.float32)
        for _ in range(3):
            y = 0.5 * (y + acc / y)
        s = jnp.exp(-3.0 * y)
        r = 1.0 / (1.0 + jnp.exp(-5.0 * (s - 0.5)))
        r_v[pl.ds(v * 16, 16)] = r
        return 0

    lax.fori_loop(0, _NTBL // 16, vec_body, 0)


def _make_sc_kernel(n_t, n_b, n_stim, n_dim):
    info = plsc.get_sparse_core_info()
    nw = info.num_cores * info.num_subcores
    n_pairs = n_t * n_b
    qb = n_b // 4
    n_units = n_pairs // qb
    per_tile = n_units // nw
    assert per_tile * nw == n_units and qb % 128 == 0

    mesh = plsc.VectorSubcoreMesh(core_axis_name="c", subcore_axis_name="s")

    nbuf = 6
    scratch = [pltpu.VMEM((n_stim, n_dim), jnp.float32),
               pltpu.VMEM((_NTBL,), jnp.float32)]
    scratch += [pltpu.VMEM((2, qb), jnp.int32) for _ in range(nbuf)]
    scratch += [pltpu.VMEM((qb,), jnp.float32) for _ in range(nbuf)]
    scratch += [pltpu.SemaphoreType.DMA for _ in range(2 * nbuf)]

    @functools.partial(
        pl.kernel,
        mesh=mesh,
        out_type=jax.ShapeDtypeStruct((n_t, 1, n_b), jnp.float32),
        compiler_params=pltpu.CompilerParams(needs_layout_passes=False),
        scratch_types=scratch,
    )
    def sc_kernel(e_hbm, x_hbm, out_hbm, e_v, r_v, *bufs):
        in_v = bufs[:nbuf]
        out_v = bufs[nbuf:2 * nbuf]
        in_sem = bufs[2 * nbuf:3 * nbuf]
        out_sem = bufs[3 * nbuf:4 * nbuf]
        wid = lax.axis_index("s") * info.num_cores + lax.axis_index("c")
        u0 = wid * per_tile

        def start_in(i):
            u = u0 + i
            return pltpu.async_copy(
                x_hbm.at[u >> 2, :, pl.ds((u & 3) * qb, qb)],
                in_v[i % nbuf], in_sem[i % nbuf])

        in_h = {}
        out_h = {}
        for i in range(min(nbuf, per_tile)):
            in_h[i] = start_in(i)

        pltpu.sync_copy(e_hbm, e_v)
        _build_rating_table(e_v, r_v, n_stim, n_dim)

        for i in range(per_tile):
            b = i % nbuf
            in_h.pop(i).wait()
            if i >= nbuf:
                out_h.pop(i - nbuf).wait()
            iv = in_v[b]
            ov = out_v[b]

            @plsc.parallel_loop(0, qb, 16, unroll=16)
            def vec_body(j):
                v0 = iv[0, pl.ds(j, 16)]
                v1 = iv[1, pl.ds(j, 16)]
                comb = ((v0 << 5) + v1) & (_NTBL - 1)
                ov[pl.ds(j, 16)] = plsc.load_gather(r_v, [comb])

            u = u0 + i
            out_h[i] = pltpu.async_copy(
                ov, out_hbm.at[u >> 2, 0, pl.ds((u & 3) * qb, qb)],
                out_sem[b])
            if i + nbuf < per_tile:
                in_h[i + nbuf] = start_in(i + nbuf)
        for i in sorted(out_h):
            out_h.pop(i).wait()

    return sc_kernel


def kernel(stimulus_set, percept_embeddings):
    b, t, two = stimulus_set.shape
    n_stim, n_dim = percept_embeddings.shape
    x3 = jnp.transpose(stimulus_set, (1, 2, 0))
    out = _make_sc_kernel(t, b, n_stim, n_dim)(percept_embeddings, x3)
    return jnp.transpose(out, (2, 0, 1))


# speedup vs baseline: 1.2466x; 1.2466x over previous
import functools

import jax
import jax.numpy as jnp
from jax import lax
from jax.experimental import pallas as pl
from jax.experimental.pallas import tpu as pltpu
from jax.experimental.pallas import tpu_sc as plsc

_TBL = 32
_NTBL = _TBL * _TBL


def _build_rating_table(e_v, r_v, n_stim, n_dim):
    @plsc.parallel_loop(0, _NTBL, 16, unroll=4)
    def vec_body(base):
        ent = lax.iota(jnp.int32, 16) + base
        i0 = jnp.minimum(ent >> 5, n_stim - 1)
        i1 = jnp.minimum(ent & (_TBL - 1), n_stim - 1)
        acc = jnp.zeros((16,), jnp.float32)
        for d in range(n_dim):
            dcol = jnp.full((16,), d, jnp.int32)
            z0 = plsc.load_gather(e_v, [dcol, i0])
            z1 = plsc.load_gather(e_v, [dcol, i1])
            df = z0 - z1
            acc = acc + df * df
        y = plsc.bitcast((plsc.bitcast(acc, jnp.int32) >> 1) + 0x1FBD1DF5,
                         jnp.float32)
        for _ in range(3):
            y = 0.5 * (y + acc / y)
        s = jnp.exp(-3.0 * y)
        r = 1.0 / (1.0 + jnp.exp(-5.0 * (s - 0.5)))
        r_v[pl.ds(base, 16)] = r


def _make_sc_kernel(n_t, n_b, n_stim, n_dim):
    info = plsc.get_sparse_core_info()
    nw = info.num_cores * info.num_subcores
    n_pairs = n_t * n_b
    qb = n_b // 4
    n_units = n_pairs // qb
    per_tile = n_units // nw
    assert per_tile * nw == n_units and qb % 128 == 0

    mesh = plsc.VectorSubcoreMesh(core_axis_name="c", subcore_axis_name="s")

    nbuf = 4
    scratch = [pltpu.VMEM((n_dim, n_stim), jnp.float32),
               pltpu.VMEM((_NTBL,), jnp.float32)]
    scratch += [pltpu.VMEM((2, qb), jnp.int32) for _ in range(nbuf)]
    scratch += [pltpu.VMEM((qb,), jnp.float32) for _ in range(nbuf)]
    scratch += [pltpu.SemaphoreType.DMA for _ in range(2 * nbuf + 1)]

    @functools.partial(
        pl.kernel,
        mesh=mesh,
        out_type=jax.ShapeDtypeStruct((n_t, 1, n_b), jnp.float32),
        compiler_params=pltpu.CompilerParams(needs_layout_passes=False),
        scratch_types=scratch,
    )
    def sc_kernel(e_hbm, x_hbm, out_hbm, e_v, r_v, *bufs):
        in_v = bufs[:nbuf]
        out_v = bufs[nbuf:2 * nbuf]
        in_sem = bufs[2 * nbuf:3 * nbuf]
        out_sem = bufs[3 * nbuf:4 * nbuf]
        e_sem = bufs[4 * nbuf]
        wid = lax.axis_index("s") * info.num_cores + lax.axis_index("c")
        u0 = wid * per_tile

        def start_in(i):
            u = u0 + i
            return pltpu.async_copy(
                x_hbm.at[u >> 2, :, pl.ds((u & 3) * qb, qb)],
                in_v[i % nbuf], in_sem[i % nbuf])

        e_h = pltpu.async_copy(e_hbm, e_v, e_sem)
        in_h = {}
        out_h = {}
        for i in range(min(nbuf, per_tile)):
            in_h[i] = start_in(i)

        e_h.wait()
        _build_rating_table(e_v, r_v, n_stim, n_dim)

        for i in range(per_tile):
            b = i % nbuf
            in_h.pop(i).wait()
            if i >= nbuf:
                out_h.pop(i - nbuf).wait()
            iv = in_v[b]
            ov = out_v[b]

            @plsc.parallel_loop(0, qb, 16, unroll=8)
            def vec_body(j):
                v0 = iv[0, pl.ds(j, 16)]
                v1 = iv[1, pl.ds(j, 16)]
                comb = ((v0 << 5) + v1) & (_NTBL - 1)
                ov[pl.ds(j, 16)] = plsc.load_gather(r_v, [comb])

            u = u0 + i
            out_h[i] = pltpu.async_copy(
                ov, out_hbm.at[u >> 2, 0, pl.ds((u & 3) * qb, qb)],
                out_sem[b])
            if i + nbuf < per_tile:
                in_h[i + nbuf] = start_in(i + nbuf)
        for i in sorted(out_h):
            out_h.pop(i).wait()

    return sc_kernel


def kernel(stimulus_set, percept_embeddings):
    b, t, two = stimulus_set.shape
    n_stim, n_dim = percept_embeddings.shape
    x3 = jnp.transpose(stimulus_set, (1, 2, 0))
    e_t = jnp.transpose(percept_embeddings, (1, 0))
    out = _make_sc_kernel(t, b, n_stim, n_dim)(e_t, x3)
    return jnp.transpose(out, (2, 0, 1))
